# baseline (speedup 1.0000x reference)
import functools

import jax
import jax.numpy as jnp
from jax import lax
from jax.experimental import pallas as pl
from jax.experimental.pallas import tpu as pltpu
from jax.experimental.pallas import tpu_sc as plsc

N = 10000
E = 160000
D_IN = 128
HID = 128
H2 = 64
NT = 16
NCH = 79
CPT = NCH * 128
CW = 128
NCW = CPT // CW
NB = 3
G = CW // 16
RPT = 632
RPL = N - (NT - 1) * RPT
BLK = 1000
DROW = 80


def _dense_core(nh, x, w0_ref, w1_ref, a_ref, wdp_ref, bdiff_ref, outs):
    i = pl.program_id(0)
    h_refs = outs[: 2 * nh]
    sv_ref = outs[2 * nh]
    mx_ref = outs[2 * nh + 1]
    h0 = jnp.dot(x, w0_ref[...], preferred_element_type=jnp.float32)
    h1 = jnp.dot(x, w1_ref[...], preferred_element_type=jnp.float32)
    for k in range(nh):
        h_refs[k][...] = h0[:, 64 * k:64 * k + 64]
        h_refs[nh + k][...] = h1[:, 64 * k:64 * k + 64]
    a = a_ref[...]
    sd0 = jnp.dot(h0, a[:, 0:2], preferred_element_type=jnp.float32)
    sd1 = jnp.dot(h1, a[:, 2:4], preferred_element_type=jnp.float32)
    z = jnp.dot(x, wdp_ref[...], preferred_element_type=jnp.float32)
    dz = z[:, 1:2] - z[:, 0:1] + bdiff_ref[0, 0]
    dec0 = 1.0 / (1.0 + jnp.exp(dz))
    dec1 = 1.0 - dec0
    zero2 = jnp.zeros_like(dec0)
    sv_ref[...] = jnp.concatenate(
        [sd0, sd1, dec0, dec1, zero2, zero2], axis=1)
    row = jnp.concatenate(
        [jnp.max(sd0, axis=0, keepdims=True),
         jnp.max(sd1, axis=0, keepdims=True),
         jnp.zeros((1, 4), jnp.float32)], axis=1)
    prev = jnp.where(i == 0, jnp.full((1, 8), -jnp.inf, jnp.float32),
                     mx_ref[...])
    mx_ref[...] = jnp.maximum(prev, row)


def _dense_body(nh, x_ref, w0_ref, w1_ref, a_ref, wdp_ref, bdiff_ref, *outs):
    _dense_core(nh, x_ref[...], w0_ref, w1_ref, a_ref, wdp_ref, bdiff_ref,
                outs)


def _dense2_body(nh, o0a_ref, o0b_ref, o1a_ref, o1b_ref, sv1_ref, w0_ref,
                 w1_ref, a_ref, wdp_ref, bdiff_ref, *outs):
    sv1 = sv1_ref[...]
    o0 = jnp.concatenate([o0a_ref[...], o0b_ref[...]], axis=1)
    o1 = jnp.concatenate([o1a_ref[...], o1b_ref[...]], axis=1)
    x2 = jax.nn.relu(sv1[:, 4:5] * o0 + sv1[:, 5:6] * o1)
    _dense_core(nh, x2, w0_ref, w1_ref, a_ref, wdp_ref, bdiff_ref, outs)


def _dense2_call(o0a, o0b, o1a, o1b, sv1, w0, w1, a4, wdp, bdiff, d_in,
                 d_out):
    nb = N // BLK
    nh = d_out // 64
    half_spec = pl.BlockSpec((BLK, 64), lambda i: (i, 0))
    h_shape = jax.ShapeDtypeStruct((N, 64), jnp.float32)
    return pl.pallas_call(
        functools.partial(_dense2_body, nh),
        grid=(nb,),
        in_specs=[half_spec, half_spec, half_spec, half_spec,
                  pl.BlockSpec((BLK, 8), lambda i: (i, 0)),
                  pl.BlockSpec((d_in, d_out), lambda i: (0, 0)),
                  pl.BlockSpec((d_in, d_out), lambda i: (0, 0)),
                  pl.BlockSpec((d_out, 4), lambda i: (0, 0)),
                  pl.BlockSpec((d_in, 128), lambda i: (0, 0)),
                  pl.BlockSpec(memory_space=pltpu.SMEM)],
        out_specs=[half_spec] * (2 * nh) + [
            pl.BlockSpec((BLK, 8), lambda i: (i, 0)),
            pl.BlockSpec((1, 8), lambda i: (0, 0)),
        ],
        out_shape=[h_shape] * (2 * nh) + [
            jax.ShapeDtypeStruct((N, 8), jnp.float32),
            jax.ShapeDtypeStruct((1, 8), jnp.float32),
        ],
    )(o0a, o0b, o1a, o1b, sv1, w0, w1, a4, wdp, bdiff)


def _dense_call(x, w0, w1, a4, wdp, bdiff, d_in, d_out):
    nb = N // BLK
    nh = d_out // 64
    h_spec = pl.BlockSpec((BLK, 64), lambda i: (i, 0))
    h_shape = jax.ShapeDtypeStruct((N, 64), jnp.float32)
    return pl.pallas_call(
        functools.partial(_dense_body, nh),
        grid=(nb,),
        in_specs=[
            pl.BlockSpec((BLK, d_in), lambda i: (i, 0)),
            pl.BlockSpec((d_in, d_out), lambda i: (0, 0)),
            pl.BlockSpec((d_in, d_out), lambda i: (0, 0)),
            pl.BlockSpec((d_out, 4), lambda i: (0, 0)),
            pl.BlockSpec((d_in, 128), lambda i: (0, 0)),
            pl.BlockSpec(memory_space=pltpu.SMEM),
        ],
        out_specs=[h_spec] * (2 * nh) + [
            pl.BlockSpec((BLK, 8), lambda i: (i, 0)),
            pl.BlockSpec((1, 8), lambda i: (0, 0)),
        ],
        out_shape=[h_shape] * (2 * nh) + [
            jax.ShapeDtypeStruct((N, 8), jnp.float32),
            jax.ShapeDtypeStruct((1, 8), jnp.float32),
        ],
    )(x, w0, w1, a4, wdp, bdiff)


def _head_body(o0_ref, o1_ref, sv_ref, f1w_ref, f1b_ref, f2w_ref, f2b_ref,
               out_ref):
    sv = sv_ref[...]
    x3 = jax.nn.relu(sv[:, 4:5] * o0_ref[...] + sv[:, 5:6] * o1_ref[...])
    z1 = jax.nn.relu(jnp.dot(x3, f1w_ref[...],
                             preferred_element_type=jnp.float32)
                     + f1b_ref[...])
    z2 = jnp.dot(z1, f2w_ref[...], preferred_element_type=jnp.float32) \
        + f2b_ref[...]
    m = jnp.max(z2, axis=1, keepdims=True)
    p = jnp.exp(z2 - m)
    out_ref[...] = p / jnp.sum(p, axis=1, keepdims=True)


def _head_call(o0, o1, sv, f1w, f1b, f2w, f2b):
    nb = N // BLK
    h4 = f1w.shape[1]
    nout = f2w.shape[1]
    return pl.pallas_call(
        _head_body,
        grid=(nb,),
        in_specs=[
            pl.BlockSpec((BLK, H2), lambda i: (i, 0)),
            pl.BlockSpec((BLK, H2), lambda i: (i, 0)),
            pl.BlockSpec((BLK, 8), lambda i: (i, 0)),
            pl.BlockSpec((H2, h4), lambda i: (0, 0)),
            pl.BlockSpec((1, h4), lambda i: (0, 0)),
            pl.BlockSpec((h4, nout), lambda i: (0, 0)),
            pl.BlockSpec((1, nout), lambda i: (0, 0)),
        ],
        out_specs=pl.BlockSpec((BLK, nout), lambda i: (i, 0)),
        out_shape=jax.ShapeDtypeStruct((N, nout), jnp.float32),
    )(o0, o1, sv, f1w, f1b, f2w, f2b)


def _sc_body(nh, *refs):
    hs = refs[0:2 * nh]
    svec, carr, srcp, dstp, zout, zden = refs[2 * nh:2 * nh + 6]
    outs = refs[2 * nh + 6:4 * nh + 6]
    rest = refs[4 * nh + 6:]
    (sv_s, sv_d, cbuf, srcv, dstv, numv, denp, iotav) = rest[:8]
    bufs = rest[8:8 + NB]
    out_sh, den_sh = rest[8 + NB:10 + NB]
    gsems = rest[10 + NB:10 + 2 * NB]
    csems = rest[10 + 2 * NB:10 + 3 * NB]

    r = lax.axis_index("c")
    s = lax.axis_index("s")

    pltpu.sync_copy(svec.at[pl.ds((2 * r) * N, N)], sv_s)
    pltpu.sync_copy(svec.at[pl.ds((2 * r + 1) * N, N)], sv_d)
    pltpu.sync_copy(carr.at[pl.ds(r * 16, 16)], cbuf)
    pltpu.sync_copy(srcp.at[r, s], srcv)
    pltpu.sync_copy(dstp.at[r, s], dstv)
    pltpu.sync_copy(zden, denp)

    @pl.when(s == 0)
    def _():
        pltpu.sync_copy(zden, den_sh)

    for k in range(DROW // 16):
        iotav[0, pl.ds(k * 16, 16)] = lax.iota(jnp.int32, 16) + (k * 16)

    cvec = cbuf[...]

    def p1_step(t, _):
        j = t // G
        k = t % G
        src16 = srcv[j, pl.ds(k * 16, 16)]
        dst16 = dstv[j, pl.ds(k * 16, 16)]
        s16 = plsc.load_gather(sv_s, [src16])
        d16 = plsc.load_gather(sv_d, [dst16])
        tt = s16 + d16
        e = jnp.where(tt > 0, tt, 0.2 * tt)
        num = jnp.exp(e - cvec)
        off = t * 16 + lax.iota(jnp.int32, 16)
        valid = (s * CPT + off) < E
        num = jnp.where(valid, num, 0.0)
        numv[j, pl.ds(k * 16, 16)] = num
        plsc.addupdate_scatter(denp, [dst16 >> 7, dst16 & 127], num)
        return 0

    lax.fori_loop(0, NCW * G, p1_step, 0, unroll=4)

    plsc.subcore_barrier()
    pltpu.sync_copy(denp, den_sh.at[iotav.at[0]], add=True)
    plsc.subcore_barrier()
    pltpu.sync_copy(den_sh, denp)

    def p2_step(t, _):
        j = t // G
        k = t % G
        dst16 = dstv[j, pl.ds(k * 16, 16)]
        den16 = plsc.load_gather(denp, [dst16 >> 7, dst16 & 127])
        num16 = numv[j, pl.ds(k * 16, 16)]
        numv[j, pl.ds(k * 16, 16)] = num16 / (den16 + 1e-16)
        return 0

    lax.fori_loop(0, NCW * G, p2_step, 0, unroll=4)

    for half in range(nh):
        @pl.when(s < NT - 1)
        def _():
            pltpu.sync_copy(zout, out_sh.at[pl.ds(s * RPT, RPT)])

        @pl.when(s == NT - 1)
        def _():
            pltpu.sync_copy(zout.at[pl.ds(0, RPL)],
                            out_sh.at[pl.ds(s * RPT, RPL)])

        plsc.subcore_barrier()

        def issue_gather(j, buf, sem):
            @pl.when(r == 0)
            def _():
                pltpu.async_copy(hs[half].at[srcv.at[j]], buf, sem)

            @pl.when(r == 1)
            def _():
                pltpu.async_copy(hs[nh + half].at[srcv.at[j]], buf, sem)

        def wait_gather(j, buf, sem):
            @pl.when(r == 0)
            def _():
                pltpu.make_async_copy(hs[half].at[srcv.at[j]], buf,
                                      sem).wait()

            @pl.when(r == 1)
            def _():
                pltpu.make_async_copy(hs[nh + half].at[srcv.at[j]], buf,
                                      sem).wait()

        def wait_scatter(j, buf, sem):
            pltpu.make_async_copy(buf, out_sh.at[dstv.at[j]], sem).wait()

        def process(j, buf, sem, scsem):
            wait_gather(j, buf, sem)

            def grp_step(k, _):
                a16 = numv[j, pl.ds(k * 16, 16)]

                def row_sub(i, _):
                    e = k * 16 + i
                    ab16 = lax.gather(
                        a16, jnp.full((16, 1), i, jnp.int32),
                        lax.GatherDimensionNumbers(
                            offset_dims=(), collapsed_slice_dims=(0,),
                            start_index_map=(0,)),
                        (1,), mode=lax.GatherScatterMode.PROMISE_IN_BOUNDS,
                    )
                    for q in range(64 // 16):
                        buf[e, pl.ds(q * 16, 16)] = \
                            buf[e, pl.ds(q * 16, 16)] * ab16
                    return 0

                lax.fori_loop(0, 16, row_sub, 0, unroll=8)
                return 0

            lax.fori_loop(0, G, grp_step, 0)
            pltpu.async_copy(buf, out_sh.at[dstv.at[j]], scsem, add=True)

        for b in range(NB - 1):
            issue_gather(b, bufs[b], gsems[b])

        def p3_chunk(j, _):
            for b in range(NB):
                @pl.when(j % NB == b)
                def _(b=b):
                    nb_ = (b + NB - 1) % NB

                    @pl.when(j + NB - 1 < NCW)
                    def _():
                        @pl.when(j >= 1)
                        def _():
                            wait_scatter(j - 1, bufs[nb_], csems[nb_])
                        issue_gather(j + NB - 1, bufs[nb_], gsems[nb_])
                    process(j, bufs[b], gsems[b], csems[b])

            return 0

        lax.fori_loop(0, NCW, p3_chunk, 0)
        for t in range(NB):
            jj = NCW - NB + t
            wait_scatter(jj, bufs[jj % NB], csems[jj % NB])
        plsc.subcore_barrier()

        for rr in range(2):
            @pl.when(jnp.logical_and(r == rr, s < NT - 1))
            def _():
                pltpu.sync_copy(out_sh.at[pl.ds(s * RPT, RPT)],
                                outs[rr * nh + half].at[pl.ds(s * RPT, RPT)])

            @pl.when(jnp.logical_and(r == rr, s == NT - 1))
            def _():
                pltpu.sync_copy(out_sh.at[pl.ds(s * RPT, RPL)],
                                outs[rr * nh + half].at[pl.ds(s * RPT, RPL)])


def _sc_call(nh, hs, svec, carr, srcp, dstp, zout, zden):
    mesh = plsc.VectorSubcoreMesh(core_axis_name="c", subcore_axis_name="s",
                                  num_cores=2, num_subcores=NT)
    f = pl.kernel(
        functools.partial(_sc_body, nh),
        out_type=[jax.ShapeDtypeStruct((N, 64), jnp.float32)] * (2 * nh),
        mesh=mesh,
        compiler_params=pltpu.CompilerParams(needs_layout_passes=False,
                                             use_tc_tiling_on_sc=False),
        scratch_types=[
            pltpu.VMEM((N,), jnp.float32),
            pltpu.VMEM((N,), jnp.float32),
            pltpu.VMEM((16,), jnp.float32),
            pltpu.VMEM((NCW, CW), jnp.int32),
            pltpu.VMEM((NCW, CW), jnp.int32),
            pltpu.VMEM((NCW, CW), jnp.float32),
            pltpu.VMEM((DROW, 128), jnp.float32),
            pltpu.VMEM((1, DROW), jnp.int32),
        ] + [pltpu.VMEM((CW, 64), jnp.float32)] * NB + [
            pltpu.VMEM_SHARED((N, 64), jnp.float32),
            pltpu.VMEM_SHARED((DROW, 128), jnp.float32),
        ] + [pltpu.SemaphoreType.DMA] * (2 * NB),
    )
    return f(*hs, svec, carr, srcp, dstp, zout, zden)


def _pad_edges(ei):
    pad = NT * CPT - E
    z = jnp.zeros((pad,), jnp.int32)
    srcp = jnp.concatenate([ei[0], z]).reshape(NT, NCW, CW)
    dstp = jnp.concatenate([ei[1], z]).reshape(NT, NCW, CW)
    return srcp, dstp


def _bound(mx, lo):
    t = mx[0, lo] + mx[0, lo + 1]
    return jnp.where(t > 0, t, 0.2 * t)


def kernel(x, edge_index_0, edge_index_1, W1_0, a1s_0, a1d_0, W1_1, a1s_1,
           a1d_1, Wd1, bd1, W2_0, a2s_0, a2d_0, W2_1, a2s_1, a2d_1, Wd2,
           bd2, fc1_w, fc1_b, fc2_w, fc2_b):
    srcp0, dstp0 = _pad_edges(edge_index_0)
    srcp1, dstp1 = _pad_edges(edge_index_1)
    srcp = jnp.stack([srcp0, srcp1])
    dstp = jnp.stack([dstp0, dstp1])

    zout = jnp.zeros((RPT, 64), jnp.float32)
    zden = jnp.zeros((DROW, 128), jnp.float32)

    a4 = jnp.stack([a1s_0, a1d_0, a1s_1, a1d_1], axis=1)
    wdp = jnp.zeros((D_IN, 128), jnp.float32).at[:, 0:2].set(Wd1)
    bdiff = (bd1[1] - bd1[0]).reshape(1, 1)
    h0a, h0b, h1a, h1b, sv, mx = _dense_call(x, W1_0, W1_1, a4, wdp, bdiff,
                                             D_IN, HID)
    svec = sv[:, 0:4].T.reshape(-1)
    carr = jnp.concatenate([jnp.full((16,), _bound(mx, 0), jnp.float32),
                            jnp.full((16,), _bound(mx, 2), jnp.float32)])
    o0a, o0b, o1a, o1b = _sc_call(2, [h0a, h0b, h1a, h1b], svec, carr,
                                  srcp, dstp, zout, zden)

    a4b = jnp.stack([a2s_0, a2d_0, a2s_1, a2d_1], axis=1)
    wdp2 = jnp.zeros((HID, 128), jnp.float32).at[:, 0:2].set(Wd2)
    bdiff2 = (bd2[1] - bd2[0]).reshape(1, 1)
    g0, g1, sv2, mx2 = _dense2_call(o0a, o0b, o1a, o1b, sv, W2_0, W2_1,
                                    a4b, wdp2, bdiff2, HID, H2)
    svec2 = sv2[:, 0:4].T.reshape(-1)
    carr2 = jnp.concatenate([jnp.full((16,), _bound(mx2, 0), jnp.float32),
                             jnp.full((16,), _bound(mx2, 2), jnp.float32)])
    p0, p1 = _sc_call(1, [g0, g1], svec2, carr2, srcp, dstp, zout, zden)

    return _head_call(p0, p1, sv2, fc1_w, fc1_b.reshape(1, -1),
                      fc2_w, fc2_b.reshape(1, -1))


# speedup vs baseline: 1.1717x; 1.1717x over previous
import functools

import jax
import jax.numpy as jnp
from jax import lax
from jax.experimental import pallas as pl
from jax.experimental.pallas import tpu as pltpu
from jax.experimental.pallas import tpu_sc as plsc

N = 10000
E = 160000
D_IN = 128
HID = 128
H2 = 64
NT = 16
NCH = 79
CPT = NCH * 128
CW = 128
NCW = CPT // CW
NB = 3
G = CW // 16
RPT = 632
RPL = N - (NT - 1) * RPT
BLK = 1000
DROW = 80


def _dense_core(nh, x, w0_ref, w1_ref, a_ref, wdp_ref, bdiff_ref, outs):
    i = pl.program_id(0)
    h_refs = outs[: 2 * nh]
    sv_ref = outs[2 * nh]
    mx_ref = outs[2 * nh + 1]
    h0 = jnp.dot(x, w0_ref[...], preferred_element_type=jnp.float32)
    h1 = jnp.dot(x, w1_ref[...], preferred_element_type=jnp.float32)
    for k in range(nh):
        h_refs[k][...] = h0[:, 64 * k:64 * k + 64]
        h_refs[nh + k][...] = h1[:, 64 * k:64 * k + 64]
    a = a_ref[...]
    sd0 = jnp.dot(h0, a[:, 0:2], preferred_element_type=jnp.float32)
    sd1 = jnp.dot(h1, a[:, 2:4], preferred_element_type=jnp.float32)
    z = jnp.dot(x, wdp_ref[...], preferred_element_type=jnp.float32)
    dz = z[:, 1:2] - z[:, 0:1] + bdiff_ref[0, 0]
    dec0 = 1.0 / (1.0 + jnp.exp(dz))
    dec1 = 1.0 - dec0
    zero2 = jnp.zeros_like(dec0)
    sv_ref[...] = jnp.concatenate(
        [sd0, sd1, dec0, dec1, zero2, zero2], axis=1)
    row = jnp.concatenate(
        [jnp.max(sd0, axis=0, keepdims=True),
         jnp.max(sd1, axis=0, keepdims=True),
         jnp.zeros((1, 4), jnp.float32)], axis=1)
    prev = jnp.where(i == 0, jnp.full((1, 8), -jnp.inf, jnp.float32),
                     mx_ref[...])
    mx_ref[...] = jnp.maximum(prev, row)


def _dense_body(nh, x_ref, w0_ref, w1_ref, a_ref, wdp_ref, bdiff_ref, *outs):
    _dense_core(nh, x_ref[...], w0_ref, w1_ref, a_ref, wdp_ref, bdiff_ref,
                outs)


def _dense2_body(nh, o0a_ref, o0b_ref, o1a_ref, o1b_ref, sv1_ref, w0_ref,
                 w1_ref, a_ref, wdp_ref, bdiff_ref, *outs):
    sv1 = sv1_ref[...]
    o0 = jnp.concatenate([o0a_ref[...], o0b_ref[...]], axis=1)
    o1 = jnp.concatenate([o1a_ref[...], o1b_ref[...]], axis=1)
    x2 = jax.nn.relu(sv1[:, 4:5] * o0 + sv1[:, 5:6] * o1)
    _dense_core(nh, x2, w0_ref, w1_ref, a_ref, wdp_ref, bdiff_ref, outs)


def _dense2_call(o0a, o0b, o1a, o1b, sv1, w0, w1, a4, wdp, bdiff, d_in,
                 d_out):
    nb = N // BLK
    nh = d_out // 64
    half_spec = pl.BlockSpec((BLK, 64), lambda i: (i, 0))
    h_shape = jax.ShapeDtypeStruct((N, 64), jnp.float32)
    return pl.pallas_call(
        functools.partial(_dense2_body, nh),
        grid=(nb,),
        in_specs=[half_spec, half_spec, half_spec, half_spec,
                  pl.BlockSpec((BLK, 8), lambda i: (i, 0)),
                  pl.BlockSpec((d_in, d_out), lambda i: (0, 0)),
                  pl.BlockSpec((d_in, d_out), lambda i: (0, 0)),
                  pl.BlockSpec((d_out, 4), lambda i: (0, 0)),
                  pl.BlockSpec((d_in, 128), lambda i: (0, 0)),
                  pl.BlockSpec(memory_space=pltpu.SMEM)],
        out_specs=[half_spec] * (2 * nh) + [
            pl.BlockSpec((BLK, 8), lambda i: (i, 0)),
            pl.BlockSpec((1, 8), lambda i: (0, 0)),
        ],
        out_shape=[h_shape] * (2 * nh) + [
            jax.ShapeDtypeStruct((N, 8), jnp.float32),
            jax.ShapeDtypeStruct((1, 8), jnp.float32),
        ],
    )(o0a, o0b, o1a, o1b, sv1, w0, w1, a4, wdp, bdiff)


def _dense_call(x, w0, w1, a4, wdp, bdiff, d_in, d_out):
    nb = N // BLK
    nh = d_out // 64
    h_spec = pl.BlockSpec((BLK, 64), lambda i: (i, 0))
    h_shape = jax.ShapeDtypeStruct((N, 64), jnp.float32)
    return pl.pallas_call(
        functools.partial(_dense_body, nh),
        grid=(nb,),
        in_specs=[
            pl.BlockSpec((BLK, d_in), lambda i: (i, 0)),
            pl.BlockSpec((d_in, d_out), lambda i: (0, 0)),
            pl.BlockSpec((d_in, d_out), lambda i: (0, 0)),
            pl.BlockSpec((d_out, 4), lambda i: (0, 0)),
            pl.BlockSpec((d_in, 128), lambda i: (0, 0)),
            pl.BlockSpec(memory_space=pltpu.SMEM),
        ],
        out_specs=[h_spec] * (2 * nh) + [
            pl.BlockSpec((BLK, 8), lambda i: (i, 0)),
            pl.BlockSpec((1, 8), lambda i: (0, 0)),
        ],
        out_shape=[h_shape] * (2 * nh) + [
            jax.ShapeDtypeStruct((N, 8), jnp.float32),
            jax.ShapeDtypeStruct((1, 8), jnp.float32),
        ],
    )(x, w0, w1, a4, wdp, bdiff)


def _head_body(o0_ref, o1_ref, sv_ref, f1w_ref, f1b_ref, f2w_ref, f2b_ref,
               out_ref):
    sv = sv_ref[...]
    x3 = jax.nn.relu(sv[:, 4:5] * o0_ref[...] + sv[:, 5:6] * o1_ref[...])
    z1 = jax.nn.relu(jnp.dot(x3, f1w_ref[...],
                             preferred_element_type=jnp.float32)
                     + f1b_ref[...])
    z2 = jnp.dot(z1, f2w_ref[...], preferred_element_type=jnp.float32) \
        + f2b_ref[...]
    m = jnp.max(z2, axis=1, keepdims=True)
    p = jnp.exp(z2 - m)
    out_ref[...] = p / jnp.sum(p, axis=1, keepdims=True)


def _head_call(o0, o1, sv, f1w, f1b, f2w, f2b):
    nb = N // BLK
    h4 = f1w.shape[1]
    nout = f2w.shape[1]
    return pl.pallas_call(
        _head_body,
        grid=(nb,),
        in_specs=[
            pl.BlockSpec((BLK, H2), lambda i: (i, 0)),
            pl.BlockSpec((BLK, H2), lambda i: (i, 0)),
            pl.BlockSpec((BLK, 8), lambda i: (i, 0)),
            pl.BlockSpec((H2, h4), lambda i: (0, 0)),
            pl.BlockSpec((1, h4), lambda i: (0, 0)),
            pl.BlockSpec((h4, nout), lambda i: (0, 0)),
            pl.BlockSpec((1, nout), lambda i: (0, 0)),
        ],
        out_specs=pl.BlockSpec((BLK, nout), lambda i: (i, 0)),
        out_shape=jax.ShapeDtypeStruct((N, nout), jnp.float32),
    )(o0, o1, sv, f1w, f1b, f2w, f2b)


def _sc_body(nh, *refs):
    hs = refs[0:2 * nh]
    svec, carr, srcp, dstp, zout, zden = refs[2 * nh:2 * nh + 6]
    outs = refs[2 * nh + 6:4 * nh + 6]
    rest = refs[4 * nh + 6:]
    (sv_s, sv_d, cbuf, srcv, dstv, numv, denp, iotav) = rest[:8]
    bufs = rest[8:8 + NB]
    out_sh, den_sh = rest[8 + NB:10 + NB]
    gsems = rest[10 + NB:10 + 2 * NB]
    csems = rest[10 + 2 * NB:10 + 3 * NB]

    r = lax.axis_index("c")
    s = lax.axis_index("s")

    pltpu.sync_copy(svec.at[pl.ds((2 * r) * N, N)], sv_s)
    pltpu.sync_copy(svec.at[pl.ds((2 * r + 1) * N, N)], sv_d)
    pltpu.sync_copy(carr.at[pl.ds(r * 16, 16)], cbuf)
    pltpu.sync_copy(srcp.at[r, s], srcv)
    pltpu.sync_copy(dstp.at[r, s], dstv)
    pltpu.sync_copy(zden, denp)

    @pl.when(s == 0)
    def _():
        pltpu.sync_copy(zden, den_sh)

    for k in range(DROW // 16):
        iotav[0, pl.ds(k * 16, 16)] = lax.iota(jnp.int32, 16) + (k * 16)

    cvec = cbuf[...]

    def p1_step(t, _):
        j = t // G
        k = t % G
        src16 = srcv[j, pl.ds(k * 16, 16)]
        dst16 = dstv[j, pl.ds(k * 16, 16)]
        s16 = plsc.load_gather(sv_s, [src16])
        d16 = plsc.load_gather(sv_d, [dst16])
        tt = s16 + d16
        e = jnp.where(tt > 0, tt, 0.2 * tt)
        num = jnp.exp(e - cvec)
        off = t * 16 + lax.iota(jnp.int32, 16)
        valid = (s * CPT + off) < E
        num = jnp.where(valid, num, 0.0)
        numv[j, pl.ds(k * 16, 16)] = num
        plsc.addupdate_scatter(denp, [dst16 >> 7, dst16 & 127], num)
        return 0

    lax.fori_loop(0, NCW * G, p1_step, 0, unroll=4)

    plsc.subcore_barrier()
    pltpu.sync_copy(denp, den_sh.at[iotav.at[0]], add=True)
    plsc.subcore_barrier()
    pltpu.sync_copy(den_sh, denp)

    def p2_step(t, _):
        j = t // G
        k = t % G
        dst16 = dstv[j, pl.ds(k * 16, 16)]
        den16 = plsc.load_gather(denp, [dst16 >> 7, dst16 & 127])
        num16 = numv[j, pl.ds(k * 16, 16)]
        numv[j, pl.ds(k * 16, 16)] = num16 / (den16 + 1e-16)
        return 0

    lax.fori_loop(0, NCW * G, p2_step, 0, unroll=4)

    for half in range(nh):
        @pl.when(s < NT - 1)
        def _():
            pltpu.sync_copy(zout, out_sh.at[pl.ds(s * RPT, RPT)])

        @pl.when(s == NT - 1)
        def _():
            pltpu.sync_copy(zout.at[pl.ds(0, RPL)],
                            out_sh.at[pl.ds(s * RPT, RPL)])

        plsc.subcore_barrier()

        def issue_gather(j, buf, sem):
            @pl.when(r == 0)
            def _():
                pltpu.async_copy(hs[half].at[srcv.at[j]], buf, sem)

            @pl.when(r == 1)
            def _():
                pltpu.async_copy(hs[nh + half].at[srcv.at[j]], buf, sem)

        def wait_gather(j, buf, sem):
            @pl.when(r == 0)
            def _():
                pltpu.make_async_copy(hs[half].at[srcv.at[j]], buf,
                                      sem).wait()

            @pl.when(r == 1)
            def _():
                pltpu.make_async_copy(hs[nh + half].at[srcv.at[j]], buf,
                                      sem).wait()

        def wait_scatter(j, buf, sem):
            pltpu.make_async_copy(buf, out_sh.at[dstv.at[j]], sem).wait()

        def process(j, buf, sem, scsem):
            wait_gather(j, buf, sem)

            def grp_step(k, _):
                a16 = numv[j, pl.ds(k * 16, 16)]

                def row_sub(i, _):
                    e = k * 16 + i
                    ab16 = lax.gather(
                        a16, jnp.full((16, 1), i, jnp.int32),
                        lax.GatherDimensionNumbers(
                            offset_dims=(), collapsed_slice_dims=(0,),
                            start_index_map=(0,)),
                        (1,), mode=lax.GatherScatterMode.PROMISE_IN_BOUNDS,
                    )
                    for q in range(64 // 16):
                        buf[e, pl.ds(q * 16, 16)] = \
                            buf[e, pl.ds(q * 16, 16)] * ab16
                    return 0

                lax.fori_loop(0, 16, row_sub, 0, unroll=4)
                return 0

            lax.fori_loop(0, G, grp_step, 0)
            pltpu.async_copy(buf, out_sh.at[dstv.at[j]], scsem, add=True)

        for b in range(NB - 1):
            issue_gather(b, bufs[b], gsems[b])

        def p3_chunk(j, _):
            for b in range(NB):
                @pl.when(j % NB == b)
                def _(b=b):
                    nb_ = (b + NB - 1) % NB

                    @pl.when(j + NB - 1 < NCW)
                    def _():
                        @pl.when(j >= 1)
                        def _():
                            wait_scatter(j - 1, bufs[nb_], csems[nb_])
                        issue_gather(j + NB - 1, bufs[nb_], gsems[nb_])
                    process(j, bufs[b], gsems[b], csems[b])

            return 0

        lax.fori_loop(0, NCW, p3_chunk, 0)
        for t in range(NB):
            jj = NCW - NB + t
            wait_scatter(jj, bufs[jj % NB], csems[jj % NB])
        plsc.subcore_barrier()

        for rr in range(2):
            @pl.when(jnp.logical_and(r == rr, s < NT - 1))
            def _():
                pltpu.sync_copy(out_sh.at[pl.ds(s * RPT, RPT)],
                                outs[rr * nh + half].at[pl.ds(s * RPT, RPT)])

            @pl.when(jnp.logical_and(r == rr, s == NT - 1))
            def _():
                pltpu.sync_copy(out_sh.at[pl.ds(s * RPT, RPL)],
                                outs[rr * nh + half].at[pl.ds(s * RPT, RPL)])


def _sc_call(nh, hs, svec, carr, srcp, dstp, zout, zden):
    mesh = plsc.VectorSubcoreMesh(core_axis_name="c", subcore_axis_name="s",
                                  num_cores=2, num_subcores=NT)
    f = pl.kernel(
        functools.partial(_sc_body, nh),
        out_type=[jax.ShapeDtypeStruct((N, 64), jnp.float32)] * (2 * nh),
        mesh=mesh,
        compiler_params=pltpu.CompilerParams(needs_layout_passes=False,
                                             use_tc_tiling_on_sc=False),
        scratch_types=[
            pltpu.VMEM((N,), jnp.float32),
            pltpu.VMEM((N,), jnp.float32),
            pltpu.VMEM((16,), jnp.float32),
            pltpu.VMEM((NCW, CW), jnp.int32),
            pltpu.VMEM((NCW, CW), jnp.int32),
            pltpu.VMEM((NCW, CW), jnp.float32),
            pltpu.VMEM((DROW, 128), jnp.float32),
            pltpu.VMEM((1, DROW), jnp.int32),
        ] + [pltpu.VMEM((CW, 64), jnp.float32)] * NB + [
            pltpu.VMEM_SHARED((N, 64), jnp.float32),
            pltpu.VMEM_SHARED((DROW, 128), jnp.float32),
        ] + [pltpu.SemaphoreType.DMA] * (2 * NB),
    )
    return f(*hs, svec, carr, srcp, dstp, zout, zden)


def _pad_edges(ei):
    pad = NT * CPT - E
    z = jnp.zeros((pad,), jnp.int32)
    srcp = jnp.concatenate([ei[0], z]).reshape(NT, NCW, CW)
    dstp = jnp.concatenate([ei[1], z]).reshape(NT, NCW, CW)
    return srcp, dstp


def _bound(mx, lo):
    t = mx[0, lo] + mx[0, lo + 1]
    return jnp.where(t > 0, t, 0.2 * t)


def kernel(x, edge_index_0, edge_index_1, W1_0, a1s_0, a1d_0, W1_1, a1s_1,
           a1d_1, Wd1, bd1, W2_0, a2s_0, a2d_0, W2_1, a2s_1, a2d_1, Wd2,
           bd2, fc1_w, fc1_b, fc2_w, fc2_b):
    srcp0, dstp0 = _pad_edges(edge_index_0)
    srcp1, dstp1 = _pad_edges(edge_index_1)
    srcp = jnp.stack([srcp0, srcp1])
    dstp = jnp.stack([dstp0, dstp1])

    zout = jnp.zeros((RPT, 64), jnp.float32)
    zden = jnp.zeros((DROW, 128), jnp.float32)

    a4 = jnp.stack([a1s_0, a1d_0, a1s_1, a1d_1], axis=1)
    wdp = jnp.zeros((D_IN, 128), jnp.float32).at[:, 0:2].set(Wd1)
    bdiff = (bd1[1] - bd1[0]).reshape(1, 1)
    h0a, h0b, h1a, h1b, sv, mx = _dense_call(x, W1_0, W1_1, a4, wdp, bdiff,
                                             D_IN, HID)
    svec = sv[:, 0:4].T.reshape(-1)
    carr = jnp.concatenate([jnp.full((16,), _bound(mx, 0), jnp.float32),
                            jnp.full((16,), _bound(mx, 2), jnp.float32)])
    o0a, o0b, o1a, o1b = _sc_call(2, [h0a, h0b, h1a, h1b], svec, carr,
                                  srcp, dstp, zout, zden)

    a4b = jnp.stack([a2s_0, a2d_0, a2s_1, a2d_1], axis=1)
    wdp2 = jnp.zeros((HID, 128), jnp.float32).at[:, 0:2].set(Wd2)
    bdiff2 = (bd2[1] - bd2[0]).reshape(1, 1)
    g0, g1, sv2, mx2 = _dense2_call(o0a, o0b, o1a, o1b, sv, W2_0, W2_1,
                                    a4b, wdp2, bdiff2, HID, H2)
    svec2 = sv2[:, 0:4].T.reshape(-1)
    carr2 = jnp.concatenate([jnp.full((16,), _bound(mx2, 0), jnp.float32),
                             jnp.full((16,), _bound(mx2, 2), jnp.float32)])
    p0, p1 = _sc_call(1, [g0, g1], svec2, carr2, srcp, dstp, zout, zden)

    return _head_call(p0, p1, sv2, fc1_w, fc1_b.reshape(1, -1),
                      fc2_w, fc2_b.reshape(1, -1))


# speedup vs baseline: 1.1751x; 1.0029x over previous
import functools

import jax
import jax.numpy as jnp
from jax import lax
from jax.experimental import pallas as pl
from jax.experimental.pallas import tpu as pltpu
from jax.experimental.pallas import tpu_sc as plsc

N = 10000
E = 160000
D_IN = 128
HID = 128
H2 = 64
NT = 16
NCH = 79
CPT = NCH * 128
CW = 128
NCW = CPT // CW
NB = 3
G = CW // 16
RPT = 632
RPL = N - (NT - 1) * RPT
BLK = 1000
DROW = 80


def _dense_core(nh, x, w0_ref, w1_ref, a_ref, wdp_ref, bdiff_ref, outs):
    i = pl.program_id(0)
    h_refs = outs[: 2 * nh]
    sv_ref = outs[2 * nh]
    mx_ref = outs[2 * nh + 1]
    h0 = jnp.dot(x, w0_ref[...], preferred_element_type=jnp.float32)
    h1 = jnp.dot(x, w1_ref[...], preferred_element_type=jnp.float32)
    for k in range(nh):
        h_refs[k][...] = h0[:, 64 * k:64 * k + 64]
        h_refs[nh + k][...] = h1[:, 64 * k:64 * k + 64]
    a = a_ref[...]
    sd0 = jnp.dot(h0, a[:, 0:2], preferred_element_type=jnp.float32)
    sd1 = jnp.dot(h1, a[:, 2:4], preferred_element_type=jnp.float32)
    z = jnp.dot(x, wdp_ref[...], preferred_element_type=jnp.float32)
    dz = z[:, 1:2] - z[:, 0:1] + bdiff_ref[0, 0]
    dec0 = 1.0 / (1.0 + jnp.exp(dz))
    dec1 = 1.0 - dec0
    zero2 = jnp.zeros_like(dec0)
    sv_ref[...] = jnp.concatenate(
        [sd0, sd1, dec0, dec1, zero2, zero2], axis=1)
    row = jnp.concatenate(
        [jnp.max(sd0, axis=0, keepdims=True),
         jnp.max(sd1, axis=0, keepdims=True),
         jnp.zeros((1, 4), jnp.float32)], axis=1)
    prev = jnp.where(i == 0, jnp.full((1, 8), -jnp.inf, jnp.float32),
                     mx_ref[...])
    mx_ref[...] = jnp.maximum(prev, row)


def _dense_body(nh, x_ref, w0_ref, w1_ref, a_ref, wdp_ref, bdiff_ref, *outs):
    _dense_core(nh, x_ref[...], w0_ref, w1_ref, a_ref, wdp_ref, bdiff_ref,
                outs)


def _dense2_body(nh, o0a_ref, o0b_ref, o1a_ref, o1b_ref, sv1_ref, w0_ref,
                 w1_ref, a_ref, wdp_ref, bdiff_ref, *outs):
    sv1 = sv1_ref[...]
    o0 = jnp.concatenate([o0a_ref[...], o0b_ref[...]], axis=1)
    o1 = jnp.concatenate([o1a_ref[...], o1b_ref[...]], axis=1)
    x2 = jax.nn.relu(sv1[:, 4:5] * o0 + sv1[:, 5:6] * o1)
    _dense_core(nh, x2, w0_ref, w1_ref, a_ref, wdp_ref, bdiff_ref, outs)


def _dense2_call(o0a, o0b, o1a, o1b, sv1, w0, w1, a4, wdp, bdiff, d_in,
                 d_out):
    nb = N // BLK
    nh = d_out // 64
    half_spec = pl.BlockSpec((BLK, 64), lambda i: (i, 0))
    h_shape = jax.ShapeDtypeStruct((N, 64), jnp.float32)
    return pl.pallas_call(
        functools.partial(_dense2_body, nh),
        grid=(nb,),
        in_specs=[half_spec, half_spec, half_spec, half_spec,
                  pl.BlockSpec((BLK, 8), lambda i: (i, 0)),
                  pl.BlockSpec((d_in, d_out), lambda i: (0, 0)),
                  pl.BlockSpec((d_in, d_out), lambda i: (0, 0)),
                  pl.BlockSpec((d_out, 4), lambda i: (0, 0)),
                  pl.BlockSpec((d_in, 128), lambda i: (0, 0)),
                  pl.BlockSpec(memory_space=pltpu.SMEM)],
        out_specs=[half_spec] * (2 * nh) + [
            pl.BlockSpec((BLK, 8), lambda i: (i, 0)),
            pl.BlockSpec((1, 8), lambda i: (0, 0)),
        ],
        out_shape=[h_shape] * (2 * nh) + [
            jax.ShapeDtypeStruct((N, 8), jnp.float32),
            jax.ShapeDtypeStruct((1, 8), jnp.float32),
        ],
    )(o0a, o0b, o1a, o1b, sv1, w0, w1, a4, wdp, bdiff)


def _dense_call(x, w0, w1, a4, wdp, bdiff, d_in, d_out):
    nb = N // BLK
    nh = d_out // 64
    h_spec = pl.BlockSpec((BLK, 64), lambda i: (i, 0))
    h_shape = jax.ShapeDtypeStruct((N, 64), jnp.float32)
    return pl.pallas_call(
        functools.partial(_dense_body, nh),
        grid=(nb,),
        in_specs=[
            pl.BlockSpec((BLK, d_in), lambda i: (i, 0)),
            pl.BlockSpec((d_in, d_out), lambda i: (0, 0)),
            pl.BlockSpec((d_in, d_out), lambda i: (0, 0)),
            pl.BlockSpec((d_out, 4), lambda i: (0, 0)),
            pl.BlockSpec((d_in, 128), lambda i: (0, 0)),
            pl.BlockSpec(memory_space=pltpu.SMEM),
        ],
        out_specs=[h_spec] * (2 * nh) + [
            pl.BlockSpec((BLK, 8), lambda i: (i, 0)),
            pl.BlockSpec((1, 8), lambda i: (0, 0)),
        ],
        out_shape=[h_shape] * (2 * nh) + [
            jax.ShapeDtypeStruct((N, 8), jnp.float32),
            jax.ShapeDtypeStruct((1, 8), jnp.float32),
        ],
    )(x, w0, w1, a4, wdp, bdiff)


def _head_body(o0_ref, o1_ref, sv_ref, f1w_ref, f1b_ref, f2w_ref, f2b_ref,
               out_ref):
    sv = sv_ref[...]
    x3 = jax.nn.relu(sv[:, 4:5] * o0_ref[...] + sv[:, 5:6] * o1_ref[...])
    z1 = jax.nn.relu(jnp.dot(x3, f1w_ref[...],
                             preferred_element_type=jnp.float32)
                     + f1b_ref[...])
    z2 = jnp.dot(z1, f2w_ref[...], preferred_element_type=jnp.float32) \
        + f2b_ref[...]
    m = jnp.max(z2, axis=1, keepdims=True)
    p = jnp.exp(z2 - m)
    out_ref[...] = p / jnp.sum(p, axis=1, keepdims=True)


def _head_call(o0, o1, sv, f1w, f1b, f2w, f2b):
    nb = N // BLK
    h4 = f1w.shape[1]
    nout = f2w.shape[1]
    return pl.pallas_call(
        _head_body,
        grid=(nb,),
        in_specs=[
            pl.BlockSpec((BLK, H2), lambda i: (i, 0)),
            pl.BlockSpec((BLK, H2), lambda i: (i, 0)),
            pl.BlockSpec((BLK, 8), lambda i: (i, 0)),
            pl.BlockSpec((H2, h4), lambda i: (0, 0)),
            pl.BlockSpec((1, h4), lambda i: (0, 0)),
            pl.BlockSpec((h4, nout), lambda i: (0, 0)),
            pl.BlockSpec((1, nout), lambda i: (0, 0)),
        ],
        out_specs=pl.BlockSpec((BLK, nout), lambda i: (i, 0)),
        out_shape=jax.ShapeDtypeStruct((N, nout), jnp.float32),
    )(o0, o1, sv, f1w, f1b, f2w, f2b)


def _sc_body(nh, *refs):
    hs = refs[0:2 * nh]
    svec, carr, srcp, dstp, zout, zden = refs[2 * nh:2 * nh + 6]
    outs = refs[2 * nh + 6:4 * nh + 6]
    rest = refs[4 * nh + 6:]
    (sv_s, sv_d, cbuf, srcv, dstv, numv, denp, iotav) = rest[:8]
    bufs = rest[8:8 + NB]
    out_sh, den_sh = rest[8 + NB:10 + NB]
    gsems = rest[10 + NB:10 + 2 * NB]
    csems = rest[10 + 2 * NB:10 + 3 * NB]

    r = lax.axis_index("c")
    s = lax.axis_index("s")

    pltpu.sync_copy(svec.at[pl.ds((2 * r) * N, N)], sv_s)
    pltpu.sync_copy(svec.at[pl.ds((2 * r + 1) * N, N)], sv_d)
    pltpu.sync_copy(carr.at[pl.ds(r * 16, 16)], cbuf)
    pltpu.sync_copy(srcp.at[r, s], srcv)
    pltpu.sync_copy(dstp.at[r, s], dstv)
    pltpu.sync_copy(zden, denp)

    @pl.when(s == 0)
    def _():
        pltpu.sync_copy(zden, den_sh)

    for k in range(DROW // 16):
        iotav[0, pl.ds(k * 16, 16)] = lax.iota(jnp.int32, 16) + (k * 16)

    cvec = cbuf[...]

    def p1_step(t, _):
        j = t // G
        k = t % G
        src16 = srcv[j, pl.ds(k * 16, 16)]
        dst16 = dstv[j, pl.ds(k * 16, 16)]
        s16 = plsc.load_gather(sv_s, [src16])
        d16 = plsc.load_gather(sv_d, [dst16])
        tt = s16 + d16
        e = jnp.where(tt > 0, tt, 0.2 * tt)
        num = jnp.exp(e - cvec)
        off = t * 16 + lax.iota(jnp.int32, 16)
        valid = (s * CPT + off) < E
        num = jnp.where(valid, num, 0.0)
        numv[j, pl.ds(k * 16, 16)] = num
        plsc.addupdate_scatter(denp, [dst16 >> 7, dst16 & 127], num)
        return 0

    lax.fori_loop(0, NCW * G, p1_step, 0, unroll=4)

    plsc.subcore_barrier()
    pltpu.sync_copy(denp, den_sh.at[iotav.at[0]], add=True)
    plsc.subcore_barrier()
    pltpu.sync_copy(den_sh, denp)

    for half in range(nh):
        @pl.when(s < NT - 1)
        def _():
            pltpu.sync_copy(zout, out_sh.at[pl.ds(s * RPT, RPT)])

        @pl.when(s == NT - 1)
        def _():
            pltpu.sync_copy(zout.at[pl.ds(0, RPL)],
                            out_sh.at[pl.ds(s * RPT, RPL)])

        plsc.subcore_barrier()

        def issue_gather(j, buf, sem):
            @pl.when(r == 0)
            def _():
                pltpu.async_copy(hs[half].at[srcv.at[j]], buf, sem)

            @pl.when(r == 1)
            def _():
                pltpu.async_copy(hs[nh + half].at[srcv.at[j]], buf, sem)

        def wait_gather(j, buf, sem):
            @pl.when(r == 0)
            def _():
                pltpu.make_async_copy(hs[half].at[srcv.at[j]], buf,
                                      sem).wait()

            @pl.when(r == 1)
            def _():
                pltpu.make_async_copy(hs[nh + half].at[srcv.at[j]], buf,
                                      sem).wait()

        def wait_scatter(j, buf, sem):
            pltpu.make_async_copy(buf, out_sh.at[dstv.at[j]], sem).wait()

        def process(j, buf, sem, scsem):
            wait_gather(j, buf, sem)

            def grp_step(k, _):
                dstg = dstv[j, pl.ds(k * 16, 16)]
                deng = plsc.load_gather(denp, [dstg >> 7, dstg & 127])
                a16 = numv[j, pl.ds(k * 16, 16)] / (deng + 1e-16)

                def row_sub(i, _):
                    e = k * 16 + i
                    ab16 = lax.gather(
                        a16, jnp.full((16, 1), i, jnp.int32),
                        lax.GatherDimensionNumbers(
                            offset_dims=(), collapsed_slice_dims=(0,),
                            start_index_map=(0,)),
                        (1,), mode=lax.GatherScatterMode.PROMISE_IN_BOUNDS,
                    )
                    for q in range(64 // 16):
                        buf[e, pl.ds(q * 16, 16)] = \
                            buf[e, pl.ds(q * 16, 16)] * ab16
                    return 0

                lax.fori_loop(0, 16, row_sub, 0, unroll=4)
                return 0

            lax.fori_loop(0, G, grp_step, 0)
            pltpu.async_copy(buf, out_sh.at[dstv.at[j]], scsem, add=True)

        for b in range(NB - 1):
            issue_gather(b, bufs[b], gsems[b])

        def p3_chunk(j, _):
            for b in range(NB):
                @pl.when(j % NB == b)
                def _(b=b):
                    nb_ = (b + NB - 1) % NB

                    @pl.when(j + NB - 1 < NCW)
                    def _():
                        @pl.when(j >= 1)
                        def _():
                            wait_scatter(j - 1, bufs[nb_], csems[nb_])
                        issue_gather(j + NB - 1, bufs[nb_], gsems[nb_])
                    process(j, bufs[b], gsems[b], csems[b])

            return 0

        lax.fori_loop(0, NCW, p3_chunk, 0)
        for t in range(NB):
            jj = NCW - NB + t
            wait_scatter(jj, bufs[jj % NB], csems[jj % NB])
        plsc.subcore_barrier()

        for rr in range(2):
            @pl.when(jnp.logical_and(r == rr, s < NT - 1))
            def _():
                pltpu.sync_copy(out_sh.at[pl.ds(s * RPT, RPT)],
                                outs[rr * nh + half].at[pl.ds(s * RPT, RPT)])

            @pl.when(jnp.logical_and(r == rr, s == NT - 1))
            def _():
                pltpu.sync_copy(out_sh.at[pl.ds(s * RPT, RPL)],
                                outs[rr * nh + half].at[pl.ds(s * RPT, RPL)])


def _sc_call(nh, hs, svec, carr, srcp, dstp, zout, zden):
    mesh = plsc.VectorSubcoreMesh(core_axis_name="c", subcore_axis_name="s",
                                  num_cores=2, num_subcores=NT)
    f = pl.kernel(
        functools.partial(_sc_body, nh),
        out_type=[jax.ShapeDtypeStruct((N, 64), jnp.float32)] * (2 * nh),
        mesh=mesh,
        compiler_params=pltpu.CompilerParams(needs_layout_passes=False,
                                             use_tc_tiling_on_sc=False),
        scratch_types=[
            pltpu.VMEM((N,), jnp.float32),
            pltpu.VMEM((N,), jnp.float32),
            pltpu.VMEM((16,), jnp.float32),
            pltpu.VMEM((NCW, CW), jnp.int32),
            pltpu.VMEM((NCW, CW), jnp.int32),
            pltpu.VMEM((NCW, CW), jnp.float32),
            pltpu.VMEM((DROW, 128), jnp.float32),
            pltpu.VMEM((1, DROW), jnp.int32),
        ] + [pltpu.VMEM((CW, 64), jnp.float32)] * NB + [
            pltpu.VMEM_SHARED((N, 64), jnp.float32),
            pltpu.VMEM_SHARED((DROW, 128), jnp.float32),
        ] + [pltpu.SemaphoreType.DMA] * (2 * NB),
    )
    return f(*hs, svec, carr, srcp, dstp, zout, zden)


def _pad_edges(ei):
    pad = NT * CPT - E
    z = jnp.zeros((pad,), jnp.int32)
    srcp = jnp.concatenate([ei[0], z]).reshape(NT, NCW, CW)
    dstp = jnp.concatenate([ei[1], z]).reshape(NT, NCW, CW)
    return srcp, dstp


def _bound(mx, lo):
    t = mx[0, lo] + mx[0, lo + 1]
    return jnp.where(t > 0, t, 0.2 * t)


def kernel(x, edge_index_0, edge_index_1, W1_0, a1s_0, a1d_0, W1_1, a1s_1,
           a1d_1, Wd1, bd1, W2_0, a2s_0, a2d_0, W2_1, a2s_1, a2d_1, Wd2,
           bd2, fc1_w, fc1_b, fc2_w, fc2_b):
    srcp0, dstp0 = _pad_edges(edge_index_0)
    srcp1, dstp1 = _pad_edges(edge_index_1)
    srcp = jnp.stack([srcp0, srcp1])
    dstp = jnp.stack([dstp0, dstp1])

    zout = jnp.zeros((RPT, 64), jnp.float32)
    zden = jnp.zeros((DROW, 128), jnp.float32)

    a4 = jnp.stack([a1s_0, a1d_0, a1s_1, a1d_1], axis=1)
    wdp = jnp.zeros((D_IN, 128), jnp.float32).at[:, 0:2].set(Wd1)
    bdiff = (bd1[1] - bd1[0]).reshape(1, 1)
    h0a, h0b, h1a, h1b, sv, mx = _dense_call(x, W1_0, W1_1, a4, wdp, bdiff,
                                             D_IN, HID)
    svec = sv[:, 0:4].T.reshape(-1)
    carr = jnp.concatenate([jnp.full((16,), _bound(mx, 0), jnp.float32),
                            jnp.full((16,), _bound(mx, 2), jnp.float32)])
    o0a, o0b, o1a, o1b = _sc_call(2, [h0a, h0b, h1a, h1b], svec, carr,
                                  srcp, dstp, zout, zden)

    a4b = jnp.stack([a2s_0, a2d_0, a2s_1, a2d_1], axis=1)
    wdp2 = jnp.zeros((HID, 128), jnp.float32).at[:, 0:2].set(Wd2)
    bdiff2 = (bd2[1] - bd2[0]).reshape(1, 1)
    g0, g1, sv2, mx2 = _dense2_call(o0a, o0b, o1a, o1b, sv, W2_0, W2_1,
                                    a4b, wdp2, bdiff2, HID, H2)
    svec2 = sv2[:, 0:4].T.reshape(-1)
    carr2 = jnp.concatenate([jnp.full((16,), _bound(mx2, 0), jnp.float32),
                             jnp.full((16,), _bound(mx2, 2), jnp.float32)])
    p0, p1 = _sc_call(1, [g0, g1], svec2, carr2, srcp, dstp, zout, zden)

    return _head_call(p0, p1, sv2, fc1_w, fc1_b.reshape(1, -1),
                      fc2_w, fc2_b.reshape(1, -1))
